# Initial kernel scaffold; baseline (speedup 1.0000x reference)
import jax
import jax.numpy as jnp
from jax.experimental import pallas as pl


def kernel(x, edge_index, W1, b1, W2, b2):
    raise NotImplementedError("write your pallas kernel here")


# speedup vs baseline: 15.8251x; 15.8251x over previous
import jax
import jax.numpy as jnp
from jax import lax
from jax.experimental import pallas as pl
from jax.experimental.pallas import tpu as pltpu
from jax.experimental.pallas import tpu_sc as plsc

N = 10000
NP = 10240
E = 320000
D = 128
NC = 2
NS = 16
NW = NC * NS
EPW = E // NW
K = 80
NCH = EPW // K
RPT = NP // NS
ZC = 128
BR = 640


def _sc_mesh():
    return plsc.VectorSubcoreMesh(
        core_axis_name="c", subcore_axis_name="s", num_cores=NC, num_subcores=NS
    )


def _deg_body(dst_hbm, ones_hbm, z128_hbm, out_hbm, didx, ones_v, dacc, sem):
    c = lax.axis_index("c")
    s = lax.axis_index("s")
    wid = s * NC + c
    row0 = s * RPT
    for j in range(5):
        pltpu.async_copy(z128_hbm, dacc.at[pl.ds(row0 + j * ZC, ZC)], sem).wait()
    pltpu.async_copy(ones_hbm, ones_v, sem).wait()
    pltpu.async_copy(dst_hbm.at[wid], didx, sem).wait()
    plsc.subcore_barrier()

    def chunk(ci, carry):
        pltpu.sync_copy(ones_v, dacc.at[didx.at[ci]], add=True)
        return carry

    lax.fori_loop(0, NCH, chunk, 0)
    plsc.subcore_barrier()
    pltpu.async_copy(dacc.at[pl.ds(row0, RPT)],
                     out_hbm.at[c].at[pl.ds(row0, RPT)], sem).wait()


def _degree(dst_r):
    ones = jnp.ones((K, D), jnp.float32)
    z128 = jnp.zeros((ZC, D), jnp.float32)
    run = pl.kernel(
        _deg_body,
        out_type=jax.ShapeDtypeStruct((NC, NP, D), jnp.float32),
        mesh=_sc_mesh(),
        scratch_types=[
            pltpu.VMEM((NCH, K), jnp.int32),
            pltpu.VMEM((K, D), jnp.float32),
            pltpu.VMEM_SHARED((NP, D), jnp.float32),
            pltpu.SemaphoreType.DMA,
        ],
    )
    return run(dst_r, ones, z128)


def _degred_body(pdeg_ref, degt_ref):
    degt_ref[...] = pdeg_ref[0, :, 0:16] + pdeg_ref[1, :, 0:16] + 1.0


def _deg_reduce(pdeg):
    return pl.pallas_call(
        _degred_body,
        grid=(NP // BR,),
        in_specs=[pl.BlockSpec((NC, BR, D), lambda i: (0, i, 0))],
        out_specs=pl.BlockSpec((BR, 16), lambda i: (i, 0)),
        out_shape=jax.ShapeDtypeStruct((NP, 16), jnp.float32),
    )(pdeg)


def _agg_body(y_hbm, src_hbm, dst_hbm, z128_hbm, out_hbm,
              sidx, didx, rows, acc, sem):
    c = lax.axis_index("c")
    s = lax.axis_index("s")
    wid = s * NC + c
    row0 = s * RPT
    for j in range(5):
        pltpu.async_copy(z128_hbm, acc.at[pl.ds(row0 + j * ZC, ZC)], sem).wait()
    pltpu.async_copy(src_hbm.at[wid], sidx, sem).wait()
    pltpu.async_copy(dst_hbm.at[wid], didx, sem).wait()
    plsc.subcore_barrier()

    def chunk(ci, carry):
        pltpu.async_copy(y_hbm.at[sidx.at[ci]], rows, sem).wait()
        pltpu.sync_copy(rows, acc.at[didx.at[ci]], add=True)
        return carry

    lax.fori_loop(0, NCH, chunk, 0)
    plsc.subcore_barrier()
    pltpu.async_copy(acc.at[pl.ds(row0, RPT)],
                     out_hbm.at[c].at[pl.ds(row0, RPT)], sem).wait()


def _aggregate(y, src_r, dst_r):
    z128 = jnp.zeros((ZC, D), jnp.float32)
    run = pl.kernel(
        _agg_body,
        out_type=jax.ShapeDtypeStruct((NC, NP, D), jnp.float32),
        mesh=_sc_mesh(),
        scratch_types=[
            pltpu.VMEM((NCH, K), jnp.int32),
            pltpu.VMEM((NCH, K), jnp.int32),
            pltpu.VMEM((K, D), jnp.float32),
            pltpu.VMEM_SHARED((NP, D), jnp.float32),
            pltpu.SemaphoreType.DMA,
        ],
    )
    return run(y, src_r, dst_r, z128)


def _mm_body(h_ref, w_ref, degt_ref, xw_ref, y_ref):
    xw = jnp.dot(h_ref[...], w_ref[...], preferred_element_type=jnp.float32)
    dinv = lax.rsqrt(degt_ref[:, 0:1])
    xw_ref[...] = xw
    y_ref[...] = xw * dinv


def _matmul_scale(h, w, degt):
    return pl.pallas_call(
        _mm_body,
        grid=(NP // BR,),
        in_specs=[
            pl.BlockSpec((BR, D), lambda i: (i, 0)),
            pl.BlockSpec((D, D), lambda i: (0, 0)),
            pl.BlockSpec((BR, 16), lambda i: (i, 0)),
        ],
        out_specs=[
            pl.BlockSpec((BR, D), lambda i: (i, 0)),
            pl.BlockSpec((BR, D), lambda i: (i, 0)),
        ],
        out_shape=[
            jax.ShapeDtypeStruct((NP, D), jnp.float32),
            jax.ShapeDtypeStruct((NP, D), jnp.float32),
        ],
    )(h, w, degt)


def _comb_body(pagg_ref, xw_ref, degt_ref, b_ref, h_ref):
    deg = degt_ref[:, 0:1]
    dinv = lax.rsqrt(deg)
    agg = pagg_ref[0] + pagg_ref[1]
    z = agg * dinv + xw_ref[...] / deg + b_ref[...]
    h_ref[...] = jnp.where(z > 0, z, jnp.exp(jnp.minimum(z, 0.0)) - 1.0)


def _combine(pagg, xw, degt, b):
    return pl.pallas_call(
        _comb_body,
        grid=(NP // BR,),
        in_specs=[
            pl.BlockSpec((NC, BR, D), lambda i: (0, i, 0)),
            pl.BlockSpec((BR, D), lambda i: (i, 0)),
            pl.BlockSpec((BR, 16), lambda i: (i, 0)),
            pl.BlockSpec((1, D), lambda i: (0, 0)),
        ],
        out_specs=pl.BlockSpec((BR, D), lambda i: (i, 0)),
        out_shape=jax.ShapeDtypeStruct((NP, D), jnp.float32),
    )(pagg, xw, degt, b)


def kernel(x, edge_index, W1, b1, W2, b2):
    src = edge_index[0].astype(jnp.int32)
    dst = edge_index[1].astype(jnp.int32)
    src_r = src.reshape(NW, NCH, K)
    dst_r = dst.reshape(NW, NCH, K)
    xp = jnp.pad(x, ((0, NP - N), (0, 0)))

    degt = _deg_reduce(_degree(dst_r))

    xw1, y1 = _matmul_scale(xp, W1, degt)
    pagg1 = _aggregate(y1, src_r, dst_r)
    h1 = _combine(pagg1, xw1, degt, b1.reshape(1, D))

    xw2, y2 = _matmul_scale(h1, W2, degt)
    pagg2 = _aggregate(y2, src_r, dst_r)
    h2 = _combine(pagg2, xw2, degt, b2.reshape(1, D))

    return jnp.concatenate([h1, h2], axis=1)[:N]


# speedup vs baseline: 20.8922x; 1.3202x over previous
import jax
import jax.numpy as jnp
from jax import lax
from jax.experimental import pallas as pl
from jax.experimental.pallas import tpu as pltpu
from jax.experimental.pallas import tpu_sc as plsc

N = 10000
NP = 10240
E = 320000
D = 128
NC = 2
NS = 16
NW = NC * NS
K = 128
NCH = 80
HCH = 40
EPW = NCH * K
EP = NW * EPW
RPT = NP // NS
ZC = 128
BR = 640


def _sc_mesh():
    return plsc.VectorSubcoreMesh(
        core_axis_name="c", subcore_axis_name="s", num_cores=NC, num_subcores=NS
    )


def _deg_body(dst_hbm, ones_hbm, z128_hbm, out_hbm, didx, ones_v, dacc, sem):
    c = lax.axis_index("c")
    s = lax.axis_index("s")
    wid = s * NC + c
    row0 = s * RPT
    for j in range(5):
        pltpu.async_copy(z128_hbm, dacc.at[pl.ds(row0 + j * ZC, ZC)], sem).wait()
    pltpu.async_copy(ones_hbm, ones_v, sem).wait()
    pltpu.async_copy(dst_hbm.at[wid], didx, sem).wait()
    plsc.subcore_barrier()

    def chunk(ci, carry):
        pltpu.sync_copy(ones_v, dacc.at[didx.at[ci]], add=True)
        return carry

    lax.fori_loop(0, NCH, chunk, 0)
    plsc.subcore_barrier()
    pltpu.async_copy(dacc.at[pl.ds(row0, RPT)],
                     out_hbm.at[c].at[pl.ds(row0, RPT)], sem).wait()


def _degree(dst_r):
    ones = jnp.ones((K, D), jnp.float32)
    z128 = jnp.zeros((ZC, D), jnp.float32)
    run = pl.kernel(
        _deg_body,
        out_type=jax.ShapeDtypeStruct((NC, NP, D), jnp.float32),
        mesh=_sc_mesh(),
        scratch_types=[
            pltpu.VMEM((NCH, K), jnp.int32),
            pltpu.VMEM((K, D), jnp.float32),
            pltpu.VMEM_SHARED((NP, D), jnp.float32),
            pltpu.SemaphoreType.DMA,
        ],
    )
    return run(dst_r, ones, z128)


def _degred_body(pdeg_ref, degt_ref):
    degt_ref[...] = pdeg_ref[0, :, 0:16] + pdeg_ref[1, :, 0:16] + 1.0


def _deg_reduce(pdeg):
    return pl.pallas_call(
        _degred_body,
        grid=(NP // BR,),
        in_specs=[pl.BlockSpec((NC, BR, D), lambda i: (0, i, 0))],
        out_specs=pl.BlockSpec((BR, 16), lambda i: (i, 0)),
        out_shape=jax.ShapeDtypeStruct((NP, 16), jnp.float32),
    )(pdeg)


def _agg_body(y_hbm, src_hbm, dst_hbm, z128_hbm, out_hbm,
              sidx, didx, rows0, rows1, acc, gsem, ssem):
    c = lax.axis_index("c")
    s = lax.axis_index("s")
    wid = s * NC + c
    row0 = s * RPT
    for j in range(5):
        pltpu.async_copy(z128_hbm, acc.at[pl.ds(row0 + j * ZC, ZC)], gsem).wait()
    plsc.subcore_barrier()

    def gather(ci, buf):
        pltpu.async_copy(y_hbm.at[sidx.at[ci]], buf, gsem)

    def wait_g():
        pltpu.make_async_copy(y_hbm.at[pl.ds(0, K)], rows0, gsem).wait()

    for h in range(NCH // HCH):
        pltpu.async_copy(src_hbm.at[wid].at[pl.ds(h * HCH, HCH)], sidx,
                         gsem).wait()
        pltpu.async_copy(dst_hbm.at[wid].at[pl.ds(h * HCH, HCH)], didx,
                         gsem).wait()
        gather(0, rows0)

        def pair(t, carry):
            a = 2 * t
            b = a + 1
            wait_g()
            gather(b, rows1)
            pltpu.sync_copy(rows0, acc.at[didx.at[a]], add=True)
            wait_g()

            @pl.when(b + 1 < HCH)
            def _():
                gather(b + 1, rows0)

            pltpu.sync_copy(rows1, acc.at[didx.at[b]], add=True)
            return carry

        lax.fori_loop(0, HCH // 2, pair, 0)
    plsc.subcore_barrier()
    pltpu.async_copy(acc.at[pl.ds(row0, RPT)],
                     out_hbm.at[c].at[pl.ds(row0, RPT)], gsem).wait()


def _aggregate(y, src_r, dst_r):
    z128 = jnp.zeros((ZC, D), jnp.float32)
    run = pl.kernel(
        _agg_body,
        out_type=jax.ShapeDtypeStruct((NC, NP, D), jnp.float32),
        mesh=_sc_mesh(),
        scratch_types=[
            pltpu.VMEM((HCH, K), jnp.int32),
            pltpu.VMEM((HCH, K), jnp.int32),
            pltpu.VMEM((K, D), jnp.float32),
            pltpu.VMEM((K, D), jnp.float32),
            pltpu.VMEM_SHARED((NP, D), jnp.float32),
            pltpu.SemaphoreType.DMA,
            pltpu.SemaphoreType.DMA,
        ],
    )
    return run(y, src_r, dst_r, z128)


def _mm_body(h_ref, w_ref, degt_ref, xw_ref, y_ref):
    xw = jnp.dot(h_ref[...], w_ref[...], preferred_element_type=jnp.float32)
    dinv = lax.rsqrt(degt_ref[:, 0:1])
    xw_ref[...] = xw
    y_ref[...] = xw * dinv


def _matmul_scale(h, w, degt):
    return pl.pallas_call(
        _mm_body,
        grid=(NP // BR,),
        in_specs=[
            pl.BlockSpec((BR, D), lambda i: (i, 0)),
            pl.BlockSpec((D, D), lambda i: (0, 0)),
            pl.BlockSpec((BR, 16), lambda i: (i, 0)),
        ],
        out_specs=[
            pl.BlockSpec((BR, D), lambda i: (i, 0)),
            pl.BlockSpec((BR, D), lambda i: (i, 0)),
        ],
        out_shape=[
            jax.ShapeDtypeStruct((NP, D), jnp.float32),
            jax.ShapeDtypeStruct((NP, D), jnp.float32),
        ],
    )(h, w, degt)


def _comb_body(pagg_ref, xw_ref, degt_ref, b_ref, h_ref):
    deg = degt_ref[:, 0:1]
    dinv = lax.rsqrt(deg)
    agg = pagg_ref[0] + pagg_ref[1]
    z = agg * dinv + xw_ref[...] / deg + b_ref[...]
    h_ref[...] = jnp.where(z > 0, z, jnp.exp(jnp.minimum(z, 0.0)) - 1.0)


def _combine(pagg, xw, degt, b):
    return pl.pallas_call(
        _comb_body,
        grid=(NP // BR,),
        in_specs=[
            pl.BlockSpec((NC, BR, D), lambda i: (0, i, 0)),
            pl.BlockSpec((BR, D), lambda i: (i, 0)),
            pl.BlockSpec((BR, 16), lambda i: (i, 0)),
            pl.BlockSpec((1, D), lambda i: (0, 0)),
        ],
        out_specs=pl.BlockSpec((BR, D), lambda i: (i, 0)),
        out_shape=jax.ShapeDtypeStruct((NP, D), jnp.float32),
    )(pagg, xw, degt, b)


def kernel(x, edge_index, W1, b1, W2, b2):
    src = edge_index[0].astype(jnp.int32)
    dst = edge_index[1].astype(jnp.int32)
    padidx = N + (jnp.arange(EP - E, dtype=jnp.int32) % (NP - N))
    src_r = jnp.concatenate([src, padidx]).reshape(NW, NCH, K)
    dst_r = jnp.concatenate([dst, padidx]).reshape(NW, NCH, K)
    xp = jnp.pad(x, ((0, NP - N), (0, 0)))

    degt = _deg_reduce(_degree(dst_r))

    xw1, y1 = _matmul_scale(xp, W1, degt)
    pagg1 = _aggregate(y1, src_r, dst_r)
    h1 = _combine(pagg1, xw1, degt, b1.reshape(1, D))

    xw2, y2 = _matmul_scale(h1, W2, degt)
    pagg2 = _aggregate(y2, src_r, dst_r)
    h2 = _combine(pagg2, xw2, degt, b2.reshape(1, D))

    return jnp.concatenate([h1, h2], axis=1)[:N]
